# baseline (speedup 1.0000x reference)
import jax
import jax.numpy as jnp
from jax import lax
from jax.experimental import pallas as pl
from jax.experimental.pallas import tpu as pltpu
from jax.experimental.pallas import tpu_sc as plsc

N = 10000
E = 320000
F = 128
NC, NS = 2, 16
NW = NC * NS
B = 128
NB = 80
EPT = NB * B
EPAD = NW * EPT
NBP = NB // 2
NA = N + 8
ZR = 624
AZC = 8
AZN = ZR // AZC
ZCH = 104
NZ = ZR // ZCH
TAIL = N - NS * ZR

_mesh = plsc.VectorSubcoreMesh(core_axis_name="c", subcore_axis_name="s")


def _deg_body(dst_hbm, out_hbm, zbuf, ones, didx, acc, isem, ssem):
    c = lax.axis_index("c")
    s = lax.axis_index("s")
    wid = s * NC + c

    icp = pltpu.async_copy(dst_hbm.at[wid], didx, isem)

    def fill(r, carry):
        zbuf[r, :] = jnp.zeros((16,), jnp.float32)
        return carry

    lax.fori_loop(0, ZCH, fill, 0)

    def fill2(r, carry):
        ones[r, :] = jnp.ones((16,), jnp.float32)
        return carry

    lax.fori_loop(0, B, fill2, 0)
    for k in range(NZ):
        pltpu.sync_copy(zbuf, acc.at[pl.ds(s * ZR + k * ZCH, ZCH)])

    @pl.when(s == 0)
    def _():
        pltpu.sync_copy(zbuf.at[pl.ds(0, TAIL)], acc.at[pl.ds(NS * ZR, TAIL)])
        pltpu.sync_copy(zbuf.at[pl.ds(0, 8)], acc.at[pl.ds(N, 8)])

    icp.wait()
    plsc.subcore_barrier()

    def step(i, carry):
        pltpu.async_copy(ones, acc.at[didx.at[i]], ssem, add=True)
        return carry

    lax.fori_loop(0, NB, step, 0)

    def drain(i, carry):
        pltpu.make_async_copy(ones, acc.at[didx.at[i]], ssem).wait()
        return carry

    lax.fori_loop(0, NB, drain, 0)
    plsc.subcore_barrier()
    pltpu.sync_copy(acc.at[pl.ds(s * ZR, ZR)], out_hbm.at[c, pl.ds(s * ZR, ZR)])

    @pl.when(s == 0)
    def _():
        pltpu.sync_copy(acc.at[pl.ds(NS * ZR, TAIL)],
                        out_hbm.at[c, pl.ds(NS * ZR, TAIL)])


_deg = pl.kernel(
    _deg_body,
    out_type=jax.ShapeDtypeStruct((NC, N, 16), jnp.float32),
    mesh=_mesh,
    scratch_types=[
        pltpu.VMEM((ZCH, 16), jnp.float32),
        pltpu.VMEM((B, 16), jnp.float32),
        pltpu.VMEM((NB, B), jnp.int32),
        pltpu.VMEM_SHARED((NA, 16), jnp.float32),
        pltpu.SemaphoreType.DMA,
        pltpu.SemaphoreType.DMA,
    ],
)


G = 8
NG = NB // G
NU = NG // 2


def _agg_body(y_hbm, src_hbm, dst_hbm, out_hbm, zbuf, sidx, dA, dB,
              r0, r1, acc, isem, zsem, dsemA, dsemB, gsem0, gsem1):
    c = lax.axis_index("c")
    s = lax.axis_index("s")
    wid = s * NC + c

    icp0 = pltpu.async_copy(src_hbm.at[wid], sidx, isem)
    pltpu.async_copy(dst_hbm.at[wid, pl.ds(0, G)], dA, dsemA)

    def fill(r, carry):
        for j in range(F // 16):
            zbuf[r, pl.ds(j * 16, 16)] = jnp.zeros((16,), jnp.float32)
        return carry

    lax.fori_loop(0, AZC, fill, 0)
    for k in range(AZN):
        pltpu.async_copy(zbuf, acc.at[pl.ds(s * ZR + k * AZC, AZC)], zsem)

    @pl.when(s == 0)
    def _():
        pltpu.async_copy(zbuf.at[pl.ds(0, TAIL)], acc.at[pl.ds(NS * ZR, TAIL)], zsem)
        pltpu.async_copy(zbuf.at[pl.ds(0, 8)], acc.at[pl.ds(N, 8)], zsem)

    for k in range(AZN):
        pltpu.make_async_copy(zbuf, acc.at[pl.ds(s * ZR + k * AZC, AZC)], zsem).wait()

    @pl.when(s == 0)
    def _():
        pltpu.make_async_copy(zbuf.at[pl.ds(0, TAIL)],
                              acc.at[pl.ds(NS * ZR, TAIL)], zsem).wait()
        pltpu.make_async_copy(zbuf.at[pl.ds(0, 8)],
                              acc.at[pl.ds(N, 8)], zsem).wait()

    icp0.wait()
    plsc.subcore_barrier()

    pltpu.async_copy(y_hbm.at[sidx.at[pl.ds(0, B)]], r0, gsem0)

    def pair(b0, dbuf, row):
        b1 = b0 + 1
        cp1 = pltpu.async_copy(y_hbm.at[sidx.at[pl.ds(b1 * B, B)]], r1, gsem1)
        pltpu.make_async_copy(y_hbm.at[sidx.at[pl.ds(b0 * B, B)]], r0, gsem0).wait()
        pltpu.sync_copy(r0, acc.at[dbuf.at[row]], add=True)

        @pl.when(b0 + 2 < NB)
        def _():
            pltpu.async_copy(y_hbm.at[sidx.at[pl.ds((b0 + 2) * B, B)]], r0, gsem0)

        cp1.wait()
        pltpu.sync_copy(r1, acc.at[dbuf.at[row + 1]], add=True)

    def super_step(u, carry):
        gbase = 2 * u * G
        pltpu.async_copy(dst_hbm.at[wid, pl.ds(gbase + G, G)], dB, dsemB)
        pltpu.make_async_copy(dst_hbm.at[wid, pl.ds(0, G)], dA, dsemA).wait()
        for k in range(G // 2):
            pair(gbase + 2 * k, dA, 2 * k)

        @pl.when(u < NU - 1)
        def _():
            pltpu.async_copy(dst_hbm.at[wid, pl.ds(gbase + 2 * G, G)], dA, dsemA)

        pltpu.make_async_copy(dst_hbm.at[wid, pl.ds(0, G)], dB, dsemB).wait()
        for k in range(G // 2):
            pair(gbase + G + 2 * k, dB, 2 * k)
        return carry

    lax.fori_loop(0, NU, super_step, 0)
    plsc.subcore_barrier()
    pltpu.sync_copy(acc.at[pl.ds(s * ZR, ZR)], out_hbm.at[c, pl.ds(s * ZR, ZR)])

    @pl.when(s == 0)
    def _():
        pltpu.sync_copy(acc.at[pl.ds(NS * ZR, TAIL)],
                        out_hbm.at[c, pl.ds(NS * ZR, TAIL)])


_agg = pl.kernel(
    _agg_body,
    out_type=jax.ShapeDtypeStruct((NC, N, F), jnp.float32),
    mesh=_mesh,
    scratch_types=[
        pltpu.VMEM((8, F), jnp.float32),
        pltpu.VMEM((EPT,), jnp.int32),
        pltpu.VMEM((G, B), jnp.int32),
        pltpu.VMEM((G, B), jnp.int32),
        pltpu.VMEM((B, F), jnp.float32),
        pltpu.VMEM((B, F), jnp.float32),
        pltpu.VMEM_SHARED((NA, F), jnp.float32),
        pltpu.SemaphoreType.DMA,
        pltpu.SemaphoreType.DMA,
        pltpu.SemaphoreType.DMA,
        pltpu.SemaphoreType.DMA,
        pltpu.SemaphoreType.DMA,
        pltpu.SemaphoreType.DMA,
    ],
)


def _tc1_body(x_ref, w_ref, degp_ref, y_ref, dinv_ref):
    deg = degp_ref[0, :, 0:1] + degp_ref[1, :, 0:1] + 1.0
    dinv = lax.rsqrt(deg)
    xw = jnp.dot(x_ref[...], w_ref[...], preferred_element_type=jnp.float32)
    y_ref[...] = xw * dinv
    dinv_ref[...] = dinv


_tc1 = pl.pallas_call(
    _tc1_body,
    out_shape=[
        jax.ShapeDtypeStruct((N, F), jnp.float32),
        jax.ShapeDtypeStruct((N, 1), jnp.float32),
    ],
)


def _tc2_body(aggp_ref, y_ref, dinv_ref, b_ref, w_ref, y2_ref):
    h = aggp_ref[0] + aggp_ref[1] + y_ref[...]
    h = jnp.maximum(h * dinv_ref[...] + b_ref[...], 0.0)
    y2_ref[...] = jnp.dot(h, w_ref[...], preferred_element_type=jnp.float32) * dinv_ref[...]


_tc2 = pl.pallas_call(
    _tc2_body,
    out_shape=jax.ShapeDtypeStruct((N, F), jnp.float32),
)


def _tc3_body(aggp_ref, y_ref, dinv_ref, b_ref, wc_ref, bc_ref, out_ref):
    h = aggp_ref[0] + aggp_ref[1] + y_ref[...]
    h = jnp.maximum(h * dinv_ref[...] + b_ref[...], 0.0)
    out_ref[...] = jnp.dot(h, wc_ref[...], preferred_element_type=jnp.float32) + bc_ref[...]


_tc3 = pl.pallas_call(
    _tc3_body,
    out_shape=jax.ShapeDtypeStruct((N, 1), jnp.float32),
)


@jax.jit
def kernel(x, edge_index, W1, b1, W2, b2, Wc, bc):
    pad = EPAD - E
    src = jnp.concatenate(
        [edge_index[0], jnp.zeros((pad,), edge_index.dtype)]).reshape(NW, EPT)
    dst = jnp.concatenate(
        [edge_index[1], jnp.full((pad,), N, edge_index.dtype)]).reshape(NW, NB, B)
    degp = _deg(dst)
    y1, dinv = _tc1(x, W1, degp)
    aggp1 = _agg(y1, src, dst)
    y2 = _tc2(aggp1, y1, dinv, b1.reshape(1, F), W2)
    aggp2 = _agg(y2, src, dst)
    out = _tc3(aggp2, y2, dinv, b2.reshape(1, F), Wc, bc.reshape(1, 1))
    return out[:, 0]


# speedup vs baseline: 1.0010x; 1.0010x over previous
import jax
import jax.numpy as jnp
from jax import lax
from jax.experimental import pallas as pl
from jax.experimental.pallas import tpu as pltpu
from jax.experimental.pallas import tpu_sc as plsc

N = 10000
E = 320000
F = 128
NC, NS = 2, 16
NW = NC * NS
B = 128
NB = 80
EPT = NB * B
EPAD = NW * EPT
NBP = NB // 2
NA = N + 8
ZR = 624
AZC = 24
AZN = ZR // AZC
ZCH = 104
NZ = ZR // ZCH
TAIL = N - NS * ZR

_mesh = plsc.VectorSubcoreMesh(core_axis_name="c", subcore_axis_name="s")


def _deg_body(dst_hbm, out_hbm, zbuf, ones, didx, acc, isem, ssem):
    c = lax.axis_index("c")
    s = lax.axis_index("s")
    wid = s * NC + c

    icp = pltpu.async_copy(dst_hbm.at[wid], didx, isem)

    def fill(r, carry):
        zbuf[r, :] = jnp.zeros((16,), jnp.float32)
        return carry

    lax.fori_loop(0, ZCH, fill, 0)

    def fill2(r, carry):
        ones[r, :] = jnp.ones((16,), jnp.float32)
        return carry

    lax.fori_loop(0, B, fill2, 0)
    for k in range(NZ):
        pltpu.sync_copy(zbuf, acc.at[pl.ds(s * ZR + k * ZCH, ZCH)])

    @pl.when(s == 0)
    def _():
        pltpu.sync_copy(zbuf.at[pl.ds(0, TAIL)], acc.at[pl.ds(NS * ZR, TAIL)])
        pltpu.sync_copy(zbuf.at[pl.ds(0, 8)], acc.at[pl.ds(N, 8)])

    icp.wait()
    plsc.subcore_barrier()

    def step(i, carry):
        pltpu.async_copy(ones, acc.at[didx.at[i]], ssem, add=True)
        return carry

    lax.fori_loop(0, NB, step, 0)

    def drain(i, carry):
        pltpu.make_async_copy(ones, acc.at[didx.at[i]], ssem).wait()
        return carry

    lax.fori_loop(0, NB, drain, 0)
    plsc.subcore_barrier()
    pltpu.sync_copy(acc.at[pl.ds(s * ZR, ZR)], out_hbm.at[c, pl.ds(s * ZR, ZR)])

    @pl.when(s == 0)
    def _():
        pltpu.sync_copy(acc.at[pl.ds(NS * ZR, TAIL)],
                        out_hbm.at[c, pl.ds(NS * ZR, TAIL)])


_deg = pl.kernel(
    _deg_body,
    out_type=jax.ShapeDtypeStruct((NC, N, 16), jnp.float32),
    mesh=_mesh,
    scratch_types=[
        pltpu.VMEM((ZCH, 16), jnp.float32),
        pltpu.VMEM((B, 16), jnp.float32),
        pltpu.VMEM((NB, B), jnp.int32),
        pltpu.VMEM_SHARED((NA, 16), jnp.float32),
        pltpu.SemaphoreType.DMA,
        pltpu.SemaphoreType.DMA,
    ],
)


G = 8
NG = NB // G
NU = NG // 2


def _agg_body(y_hbm, src_hbm, dst_hbm, out_hbm, zbuf, sidx, dA, dB,
              r0, r1, acc, isem, zsem, dsemA, dsemB, gsem0, gsem1):
    c = lax.axis_index("c")
    s = lax.axis_index("s")
    wid = s * NC + c

    icp0 = pltpu.async_copy(src_hbm.at[wid], sidx, isem)
    pltpu.async_copy(dst_hbm.at[wid, pl.ds(0, G)], dA, dsemA)

    def fill(r, carry):
        for j in range(F // 16):
            zbuf[r, pl.ds(j * 16, 16)] = jnp.zeros((16,), jnp.float32)
        return carry

    lax.fori_loop(0, AZC, fill, 0)
    for k in range(AZN):
        pltpu.async_copy(zbuf, acc.at[pl.ds(s * ZR + k * AZC, AZC)], zsem)

    @pl.when(s == 0)
    def _():
        pltpu.async_copy(zbuf, acc.at[pl.ds(NS * ZR, TAIL + 8)], zsem)

    for k in range(AZN):
        pltpu.make_async_copy(zbuf, acc.at[pl.ds(s * ZR + k * AZC, AZC)], zsem).wait()

    @pl.when(s == 0)
    def _():
        pltpu.make_async_copy(zbuf, acc.at[pl.ds(NS * ZR, TAIL + 8)], zsem).wait()

    icp0.wait()
    plsc.subcore_barrier()

    pltpu.async_copy(y_hbm.at[sidx.at[pl.ds(0, B)]], r0, gsem0)

    def pair(b0, dbuf, row):
        b1 = b0 + 1
        cp1 = pltpu.async_copy(y_hbm.at[sidx.at[pl.ds(b1 * B, B)]], r1, gsem1)
        pltpu.make_async_copy(y_hbm.at[sidx.at[pl.ds(b0 * B, B)]], r0, gsem0).wait()
        pltpu.sync_copy(r0, acc.at[dbuf.at[row]], add=True)

        @pl.when(b0 + 2 < NB)
        def _():
            pltpu.async_copy(y_hbm.at[sidx.at[pl.ds((b0 + 2) * B, B)]], r0, gsem0)

        cp1.wait()
        pltpu.sync_copy(r1, acc.at[dbuf.at[row + 1]], add=True)

    def super_step(u, carry):
        gbase = 2 * u * G
        pltpu.async_copy(dst_hbm.at[wid, pl.ds(gbase + G, G)], dB, dsemB)
        pltpu.make_async_copy(dst_hbm.at[wid, pl.ds(0, G)], dA, dsemA).wait()
        for k in range(G // 2):
            pair(gbase + 2 * k, dA, 2 * k)

        @pl.when(u < NU - 1)
        def _():
            pltpu.async_copy(dst_hbm.at[wid, pl.ds(gbase + 2 * G, G)], dA, dsemA)

        pltpu.make_async_copy(dst_hbm.at[wid, pl.ds(0, G)], dB, dsemB).wait()
        for k in range(G // 2):
            pair(gbase + G + 2 * k, dB, 2 * k)
        return carry

    lax.fori_loop(0, NU, super_step, 0)
    plsc.subcore_barrier()
    pltpu.sync_copy(acc.at[pl.ds(s * ZR, ZR)], out_hbm.at[c, pl.ds(s * ZR, ZR)])

    @pl.when(s == 0)
    def _():
        pltpu.sync_copy(acc.at[pl.ds(NS * ZR, TAIL)],
                        out_hbm.at[c, pl.ds(NS * ZR, TAIL)])


_agg = pl.kernel(
    _agg_body,
    out_type=jax.ShapeDtypeStruct((NC, N, F), jnp.float32),
    mesh=_mesh,
    scratch_types=[
        pltpu.VMEM((AZC, F), jnp.float32),
        pltpu.VMEM((EPT,), jnp.int32),
        pltpu.VMEM((G, B), jnp.int32),
        pltpu.VMEM((G, B), jnp.int32),
        pltpu.VMEM((B, F), jnp.float32),
        pltpu.VMEM((B, F), jnp.float32),
        pltpu.VMEM_SHARED((NA, F), jnp.float32),
        pltpu.SemaphoreType.DMA,
        pltpu.SemaphoreType.DMA,
        pltpu.SemaphoreType.DMA,
        pltpu.SemaphoreType.DMA,
        pltpu.SemaphoreType.DMA,
        pltpu.SemaphoreType.DMA,
    ],
)


def _tc1_body(x_ref, w_ref, degp_ref, y_ref, dinv_ref):
    deg = degp_ref[0, :, 0:1] + degp_ref[1, :, 0:1] + 1.0
    dinv = lax.rsqrt(deg)
    xw = jnp.dot(x_ref[...], w_ref[...], preferred_element_type=jnp.float32)
    y_ref[...] = xw * dinv
    dinv_ref[...] = dinv


_tc1 = pl.pallas_call(
    _tc1_body,
    out_shape=[
        jax.ShapeDtypeStruct((N, F), jnp.float32),
        jax.ShapeDtypeStruct((N, 1), jnp.float32),
    ],
)


def _tc2_body(aggp_ref, y_ref, dinv_ref, b_ref, w_ref, y2_ref):
    h = aggp_ref[0] + aggp_ref[1] + y_ref[...]
    h = jnp.maximum(h * dinv_ref[...] + b_ref[...], 0.0)
    y2_ref[...] = jnp.dot(h, w_ref[...], preferred_element_type=jnp.float32) * dinv_ref[...]


_tc2 = pl.pallas_call(
    _tc2_body,
    out_shape=jax.ShapeDtypeStruct((N, F), jnp.float32),
)


def _tc3_body(aggp_ref, y_ref, dinv_ref, b_ref, wc_ref, bc_ref, out_ref):
    h = aggp_ref[0] + aggp_ref[1] + y_ref[...]
    h = jnp.maximum(h * dinv_ref[...] + b_ref[...], 0.0)
    out_ref[...] = jnp.dot(h, wc_ref[...], preferred_element_type=jnp.float32) + bc_ref[...]


_tc3 = pl.pallas_call(
    _tc3_body,
    out_shape=jax.ShapeDtypeStruct((N, 1), jnp.float32),
)


@jax.jit
def kernel(x, edge_index, W1, b1, W2, b2, Wc, bc):
    pad = EPAD - E
    src = jnp.concatenate(
        [edge_index[0], jnp.zeros((pad,), edge_index.dtype)]).reshape(NW, EPT)
    dst = jnp.concatenate(
        [edge_index[1], jnp.full((pad,), N, edge_index.dtype)]).reshape(NW, NB, B)
    degp = _deg(dst)
    y1, dinv = _tc1(x, W1, degp)
    aggp1 = _agg(y1, src, dst)
    y2 = _tc2(aggp1, y1, dinv, b1.reshape(1, F), W2)
    aggp2 = _agg(y2, src, dst)
    out = _tc3(aggp2, y2, dinv, b2.reshape(1, F), Wc, bc.reshape(1, 1))
    return out[:, 0]


# speedup vs baseline: 1.2394x; 1.2382x over previous
import jax
import jax.numpy as jnp
from jax import lax
from jax.experimental import pallas as pl
from jax.experimental.pallas import tpu as pltpu
from jax.experimental.pallas import tpu_sc as plsc

N = 10000
E = 320000
F = 128
NC, NS = 2, 16
NW = NC * NS
B = 128
NB = 80
EPT = NB * B
EPAD = NW * EPT
NBP = NB // 2
NA = N + 8
ZR = 624
AZC = 24
AZN = ZR // AZC
ZCH = 104
NZ = ZR // ZCH
TAIL = N - NS * ZR

_mesh = plsc.VectorSubcoreMesh(core_axis_name="c", subcore_axis_name="s")


def _deg_body(dst_hbm, out_hbm, zbuf, ones, didx, acc, isem, ssem):
    c = lax.axis_index("c")
    s = lax.axis_index("s")
    wid = s * NC + c

    icp = pltpu.async_copy(dst_hbm.at[wid], didx, isem)

    def fill(r, carry):
        zbuf[r, :] = jnp.zeros((16,), jnp.float32)
        return carry

    lax.fori_loop(0, ZCH, fill, 0)

    def fill2(r, carry):
        ones[r, :] = jnp.ones((16,), jnp.float32)
        return carry

    lax.fori_loop(0, B, fill2, 0)
    for k in range(NZ):
        pltpu.sync_copy(zbuf, acc.at[pl.ds(s * ZR + k * ZCH, ZCH)])

    @pl.when(s == 0)
    def _():
        pltpu.sync_copy(zbuf.at[pl.ds(0, TAIL)], acc.at[pl.ds(NS * ZR, TAIL)])
        pltpu.sync_copy(zbuf.at[pl.ds(0, 8)], acc.at[pl.ds(N, 8)])

    icp.wait()
    plsc.subcore_barrier()

    def step(i, carry):
        pltpu.async_copy(ones, acc.at[didx.at[i]], ssem, add=True)
        return carry

    lax.fori_loop(0, NB, step, 0)

    def drain(i, carry):
        pltpu.make_async_copy(ones, acc.at[didx.at[i]], ssem).wait()
        return carry

    lax.fori_loop(0, NB, drain, 0)
    plsc.subcore_barrier()
    pltpu.sync_copy(acc.at[pl.ds(s * ZR, ZR)], out_hbm.at[c, pl.ds(s * ZR, ZR)])

    @pl.when(s == 0)
    def _():
        pltpu.sync_copy(acc.at[pl.ds(NS * ZR, TAIL)],
                        out_hbm.at[c, pl.ds(NS * ZR, TAIL)])


_deg = pl.kernel(
    _deg_body,
    out_type=jax.ShapeDtypeStruct((NC, N, 16), jnp.float32),
    mesh=_mesh,
    scratch_types=[
        pltpu.VMEM((ZCH, 16), jnp.float32),
        pltpu.VMEM((B, 16), jnp.float32),
        pltpu.VMEM((NB, B), jnp.int32),
        pltpu.VMEM_SHARED((NA, 16), jnp.float32),
        pltpu.SemaphoreType.DMA,
        pltpu.SemaphoreType.DMA,
    ],
)


G = 8
NG = NB // G
NU = NG // 2


def _agg_body(y_hbm, src_hbm, dst_hbm, out_hbm, zbuf, sidx, dA, dB,
              r0, r1, acc, isem, zsem, dsemA, dsemB, gsem0, gsem1):
    c = lax.axis_index("c")
    s = lax.axis_index("s")
    wid = s * NC + c

    icp0 = pltpu.async_copy(src_hbm.at[wid], sidx, isem)
    pltpu.async_copy(dst_hbm.at[wid, pl.ds(0, G)], dA, dsemA)

    def fill(r, carry):
        for j in range(F // 16):
            zbuf[r, pl.ds(j * 16, 16)] = jnp.zeros((16,), jnp.float32)
        return carry

    lax.fori_loop(0, AZC, fill, 0)
    for k in range(AZN):
        pltpu.async_copy(zbuf, acc.at[pl.ds(s * ZR + k * AZC, AZC)], zsem)

    @pl.when(s == 0)
    def _():
        pltpu.async_copy(zbuf, acc.at[pl.ds(NS * ZR, TAIL + 8)], zsem)

    for k in range(AZN):
        pltpu.make_async_copy(zbuf, acc.at[pl.ds(s * ZR + k * AZC, AZC)], zsem).wait()

    @pl.when(s == 0)
    def _():
        pltpu.make_async_copy(zbuf, acc.at[pl.ds(NS * ZR, TAIL + 8)], zsem).wait()

    icp0.wait()
    plsc.subcore_barrier()

    pltpu.async_copy(y_hbm.at[sidx.at[pl.ds(0, B)]], r0, gsem0)

    def pair(b0, dbuf, row):
        b1 = b0 + 1
        cp1 = pltpu.async_copy(y_hbm.at[sidx.at[pl.ds(b1 * B, B)]], r1, gsem1)
        pltpu.make_async_copy(y_hbm.at[sidx.at[pl.ds(b0 * B, B)]], r0, gsem0).wait()
        pltpu.sync_copy(r0, acc.at[dbuf.at[row]], add=True)

        @pl.when(b0 + 2 < NB)
        def _():
            pltpu.async_copy(y_hbm.at[sidx.at[pl.ds((b0 + 2) * B, B)]], r0, gsem0)

        cp1.wait()
        pltpu.sync_copy(r1, acc.at[dbuf.at[row + 1]], add=True)

    def super_step(u, carry):
        gbase = 2 * u * G
        pltpu.async_copy(dst_hbm.at[wid, pl.ds(gbase + G, G)], dB, dsemB)
        pltpu.make_async_copy(dst_hbm.at[wid, pl.ds(0, G)], dA, dsemA).wait()
        for k in range(G // 2):
            pair(gbase + 2 * k, dA, 2 * k)

        @pl.when(u < NU - 1)
        def _():
            pltpu.async_copy(dst_hbm.at[wid, pl.ds(gbase + 2 * G, G)], dA, dsemA)

        pltpu.make_async_copy(dst_hbm.at[wid, pl.ds(0, G)], dB, dsemB).wait()
        for k in range(G // 2):
            pair(gbase + G + 2 * k, dB, 2 * k)
        return carry

    lax.fori_loop(0, NU, super_step, 0)
    plsc.subcore_barrier()
    pltpu.sync_copy(acc.at[pl.ds(s * ZR, ZR)], out_hbm.at[c, pl.ds(s * ZR, ZR)])

    @pl.when(s == 0)
    def _():
        pltpu.sync_copy(acc.at[pl.ds(NS * ZR, TAIL)],
                        out_hbm.at[c, pl.ds(NS * ZR, TAIL)])


_agg = pl.kernel(
    _agg_body,
    out_type=jax.ShapeDtypeStruct((NC, N, F), jnp.float32),
    mesh=_mesh,
    scratch_types=[
        pltpu.VMEM((AZC, F), jnp.float32),
        pltpu.VMEM((EPT,), jnp.int32),
        pltpu.VMEM((G, B), jnp.int32),
        pltpu.VMEM((G, B), jnp.int32),
        pltpu.VMEM((B, F), jnp.float32),
        pltpu.VMEM((B, F), jnp.float32),
        pltpu.VMEM_SHARED((NA, F), jnp.float32),
        pltpu.SemaphoreType.DMA,
        pltpu.SemaphoreType.DMA,
        pltpu.SemaphoreType.DMA,
        pltpu.SemaphoreType.DMA,
        pltpu.SemaphoreType.DMA,
        pltpu.SemaphoreType.DMA,
    ],
)


def _tc1_body(x_ref, w_ref, degp_ref, y_ref, dinv_ref):
    deg = degp_ref[0, :, 0:1] + degp_ref[1, :, 0:1] + 1.0
    dinv = lax.rsqrt(deg)
    xw = jnp.dot(x_ref[...], w_ref[...], preferred_element_type=jnp.float32)
    y_ref[...] = xw * dinv
    dinv_ref[...] = dinv


_tc1 = pl.pallas_call(
    _tc1_body,
    out_shape=[
        jax.ShapeDtypeStruct((N, F), jnp.float32),
        jax.ShapeDtypeStruct((N, 1), jnp.float32),
    ],
)


def _tc2_body(aggp_ref, y_ref, dinv_ref, b_ref, w_ref, y2_ref):
    h = aggp_ref[0] + aggp_ref[1] + y_ref[...]
    h = jnp.maximum(h * dinv_ref[...] + b_ref[...], 0.0)
    y2_ref[...] = jnp.dot(h, w_ref[...], preferred_element_type=jnp.float32) * dinv_ref[...]


_tc2 = pl.pallas_call(
    _tc2_body,
    out_shape=jax.ShapeDtypeStruct((N, F), jnp.float32),
)


def _tc3_body(aggp_ref, y_ref, dinv_ref, b_ref, wc_ref, bc_ref, out_ref):
    h = aggp_ref[0] + aggp_ref[1] + y_ref[...]
    h = jnp.maximum(h * dinv_ref[...] + b_ref[...], 0.0)
    out_ref[...] = jnp.dot(h, wc_ref[...], preferred_element_type=jnp.float32) + bc_ref[...]


_tc3 = pl.pallas_call(
    _tc3_body,
    out_shape=jax.ShapeDtypeStruct((N, 1), jnp.float32),
)


@jax.jit
def kernel(x, edge_index, W1, b1, W2, b2, Wc, bc):
    pad = EPT - E // NW
    pad_src = jnp.zeros((NW, pad), edge_index.dtype)
    pad_dst = jnp.broadcast_to(N + (jnp.arange(pad) % 8), (NW, pad)).astype(edge_index.dtype)
    src = jnp.concatenate(
        [edge_index[0].reshape(NW, E // NW), pad_src], axis=1).reshape(NW, EPT)
    dst = jnp.concatenate(
        [edge_index[1].reshape(NW, E // NW), pad_dst], axis=1).reshape(NW, NB, B)
    degp = _deg(dst)
    y1, dinv = _tc1(x, W1, degp)
    aggp1 = _agg(y1, src, dst)
    y2 = _tc2(aggp1, y1, dinv, b1.reshape(1, F), W2)
    aggp2 = _agg(y2, src, dst)
    out = _tc3(aggp2, y2, dinv, b2.reshape(1, F), Wc, bc.reshape(1, 1))
    return out[:, 0]


# speedup vs baseline: 1.8479x; 1.4910x over previous
import jax
import jax.numpy as jnp
from jax import lax
from jax.experimental import pallas as pl
from jax.experimental.pallas import tpu as pltpu
from jax.experimental.pallas import tpu_sc as plsc

N = 10000
E = 320000
F = 128
NC, NS = 2, 16
NW = NC * NS
B = 128
NB = 80
EPT = NB * B
EPAD = NW * EPT
NBP = NB // 2
NA = N + 8
ZR = 624
AZC = 24
AZN = ZR // AZC
ZCH = 104
NZ = ZR // ZCH
TAIL = N - NS * ZR

_mesh = plsc.VectorSubcoreMesh(core_axis_name="c", subcore_axis_name="s")


def _deg_body(dst_hbm, out_hbm, zbuf, ones, didx, acc, isem, ssem):
    c = lax.axis_index("c")
    s = lax.axis_index("s")
    wid = s * NC + c

    icp = pltpu.async_copy(dst_hbm.at[wid], didx, isem)

    def fill(r, carry):
        zbuf[r, :] = jnp.zeros((16,), jnp.float32)
        return carry

    lax.fori_loop(0, ZCH, fill, 0)

    def fill2(r, carry):
        ones[r, :] = jnp.ones((16,), jnp.float32)
        return carry

    lax.fori_loop(0, B, fill2, 0)
    for k in range(NZ):
        pltpu.sync_copy(zbuf, acc.at[pl.ds(s * ZR + k * ZCH, ZCH)])

    @pl.when(s == 0)
    def _():
        pltpu.sync_copy(zbuf.at[pl.ds(0, TAIL)], acc.at[pl.ds(NS * ZR, TAIL)])
        pltpu.sync_copy(zbuf.at[pl.ds(0, 8)], acc.at[pl.ds(N, 8)])

    icp.wait()
    plsc.subcore_barrier()

    def step(i, carry):
        pltpu.async_copy(ones, acc.at[didx.at[i]], ssem, add=True)
        return carry

    lax.fori_loop(0, NB, step, 0)

    def drain(i, carry):
        pltpu.make_async_copy(ones, acc.at[didx.at[i]], ssem).wait()
        return carry

    lax.fori_loop(0, NB, drain, 0)
    plsc.subcore_barrier()
    pltpu.sync_copy(acc.at[pl.ds(s * ZR, ZR)], out_hbm.at[c, pl.ds(s * ZR, ZR)])

    @pl.when(s == 0)
    def _():
        pltpu.sync_copy(acc.at[pl.ds(NS * ZR, TAIL)],
                        out_hbm.at[c, pl.ds(NS * ZR, TAIL)])


_deg = pl.kernel(
    _deg_body,
    out_type=jax.ShapeDtypeStruct((NC, N, 16), jnp.float32),
    mesh=_mesh,
    scratch_types=[
        pltpu.VMEM((ZCH, 16), jnp.float32),
        pltpu.VMEM((B, 16), jnp.float32),
        pltpu.VMEM((NB, B), jnp.int32),
        pltpu.VMEM_SHARED((NA, 16), jnp.float32),
        pltpu.SemaphoreType.DMA,
        pltpu.SemaphoreType.DMA,
    ],
)


G = 8
NG = NB // G
NU = NG // 2


def _agg_body(y_hbm, src_hbm, dst_hbm, out_hbm, zbuf, sidx, dA, dB,
              r0, r1, acc, isem, zsem, dsemA, dsemB, gsem0, gsem1):
    c = lax.axis_index("c")
    s = lax.axis_index("s")
    wid = s * NC + c

    icp0 = pltpu.async_copy(src_hbm.at[wid], sidx, isem)
    pltpu.async_copy(dst_hbm.at[wid, pl.ds(0, G)], dA, dsemA)

    def fill(r, carry):
        for j in range(F // 16):
            zbuf[r, pl.ds(j * 16, 16)] = jnp.zeros((16,), jnp.float32)
        return carry

    lax.fori_loop(0, AZC, fill, 0)
    for k in range(AZN):
        pltpu.async_copy(zbuf, acc.at[pl.ds(s * ZR + k * AZC, AZC)], zsem)

    @pl.when(s == 0)
    def _():
        pltpu.async_copy(zbuf, acc.at[pl.ds(NS * ZR, TAIL + 8)], zsem)

    for k in range(AZN):
        pltpu.make_async_copy(zbuf, acc.at[pl.ds(s * ZR + k * AZC, AZC)], zsem).wait()

    @pl.when(s == 0)
    def _():
        pltpu.make_async_copy(zbuf, acc.at[pl.ds(NS * ZR, TAIL + 8)], zsem).wait()

    icp0.wait()
    plsc.subcore_barrier()

    pltpu.async_copy(y_hbm.at[c].at[sidx.at[pl.ds(0, B)]], r0, gsem0)

    def pair(b0, dbuf, row):
        b1 = b0 + 1
        cp1 = pltpu.async_copy(y_hbm.at[c].at[sidx.at[pl.ds(b1 * B, B)]], r1, gsem1)
        pltpu.make_async_copy(y_hbm.at[c].at[sidx.at[pl.ds(b0 * B, B)]], r0, gsem0).wait()
        pltpu.sync_copy(r0, acc.at[dbuf.at[row]], add=True)

        @pl.when(b0 + 2 < NB)
        def _():
            pltpu.async_copy(y_hbm.at[c].at[sidx.at[pl.ds((b0 + 2) * B, B)]], r0, gsem0)

        cp1.wait()
        pltpu.sync_copy(r1, acc.at[dbuf.at[row + 1]], add=True)

    def super_step(u, carry):
        gbase = 2 * u * G
        pltpu.async_copy(dst_hbm.at[wid, pl.ds(gbase + G, G)], dB, dsemB)
        pltpu.make_async_copy(dst_hbm.at[wid, pl.ds(0, G)], dA, dsemA).wait()
        for k in range(G // 2):
            pair(gbase + 2 * k, dA, 2 * k)

        @pl.when(u < NU - 1)
        def _():
            pltpu.async_copy(dst_hbm.at[wid, pl.ds(gbase + 2 * G, G)], dA, dsemA)

        pltpu.make_async_copy(dst_hbm.at[wid, pl.ds(0, G)], dB, dsemB).wait()
        for k in range(G // 2):
            pair(gbase + G + 2 * k, dB, 2 * k)
        return carry

    lax.fori_loop(0, NU, super_step, 0)
    plsc.subcore_barrier()
    pltpu.sync_copy(acc.at[pl.ds(s * ZR, ZR)], out_hbm.at[c, pl.ds(s * ZR, ZR)])

    @pl.when(s == 0)
    def _():
        pltpu.sync_copy(acc.at[pl.ds(NS * ZR, TAIL)],
                        out_hbm.at[c, pl.ds(NS * ZR, TAIL)])


_agg = pl.kernel(
    _agg_body,
    out_type=jax.ShapeDtypeStruct((NC, N, F), jnp.float32),
    mesh=_mesh,
    scratch_types=[
        pltpu.VMEM((AZC, F), jnp.float32),
        pltpu.VMEM((EPT,), jnp.int32),
        pltpu.VMEM((G, B), jnp.int32),
        pltpu.VMEM((G, B), jnp.int32),
        pltpu.VMEM((B, F), jnp.float32),
        pltpu.VMEM((B, F), jnp.float32),
        pltpu.VMEM_SHARED((NA, F), jnp.float32),
        pltpu.SemaphoreType.DMA,
        pltpu.SemaphoreType.DMA,
        pltpu.SemaphoreType.DMA,
        pltpu.SemaphoreType.DMA,
        pltpu.SemaphoreType.DMA,
        pltpu.SemaphoreType.DMA,
    ],
)


def _tc1_body(x_ref, w_ref, degp_ref, y_ref, dinv_ref):
    deg = degp_ref[0, :, 0:1] + degp_ref[1, :, 0:1] + 1.0
    dinv = lax.rsqrt(deg)
    xw = jnp.dot(x_ref[...], w_ref[...], preferred_element_type=jnp.float32)
    y = xw * dinv
    y_ref[0] = y
    y_ref[1] = y
    dinv_ref[...] = dinv


_tc1 = pl.pallas_call(
    _tc1_body,
    out_shape=[
        jax.ShapeDtypeStruct((NC, N, F), jnp.float32),
        jax.ShapeDtypeStruct((N, 1), jnp.float32),
    ],
)


def _tc2_body(aggp_ref, y_ref, dinv_ref, b_ref, w_ref, y2_ref):
    h = aggp_ref[0] + aggp_ref[1] + y_ref[0]
    h = jnp.maximum(h * dinv_ref[...] + b_ref[...], 0.0)
    y2 = jnp.dot(h, w_ref[...], preferred_element_type=jnp.float32) * dinv_ref[...]
    y2_ref[0] = y2
    y2_ref[1] = y2


_tc2 = pl.pallas_call(
    _tc2_body,
    out_shape=jax.ShapeDtypeStruct((NC, N, F), jnp.float32),
)


def _tc3_body(aggp_ref, y_ref, dinv_ref, b_ref, wc_ref, bc_ref, out_ref):
    h = aggp_ref[0] + aggp_ref[1] + y_ref[0]
    h = jnp.maximum(h * dinv_ref[...] + b_ref[...], 0.0)
    out_ref[...] = jnp.dot(h, wc_ref[...], preferred_element_type=jnp.float32) + bc_ref[...]


_tc3 = pl.pallas_call(
    _tc3_body,
    out_shape=jax.ShapeDtypeStruct((N, 1), jnp.float32),
)


@jax.jit
def kernel(x, edge_index, W1, b1, W2, b2, Wc, bc):
    pad = EPT - E // NW
    pad_src = jnp.zeros((NW, pad), edge_index.dtype)
    pad_dst = jnp.broadcast_to(N + (jnp.arange(pad) % 8), (NW, pad)).astype(edge_index.dtype)
    src = jnp.concatenate(
        [edge_index[0].reshape(NW, E // NW), pad_src], axis=1).reshape(NW, EPT)
    dst = jnp.concatenate(
        [edge_index[1].reshape(NW, E // NW), pad_dst], axis=1).reshape(NW, NB, B)
    degp = _deg(dst)
    y1, dinv = _tc1(x, W1, degp)
    aggp1 = _agg(y1, src, dst)
    y2 = _tc2(aggp1, y1, dinv, b1.reshape(1, F), W2)
    aggp2 = _agg(y2, src, dst)
    out = _tc3(aggp2, y2, dinv, b2.reshape(1, F), Wc, bc.reshape(1, 1))
    return out[:, 0]
